# baseline (speedup 1.0000x reference)
import functools

import jax
import jax.numpy as jnp
from jax import lax
from jax.experimental import pallas as pl
from jax.experimental.pallas import tpu as pltpu
from jax.experimental.pallas import tpu_sc as plsc

VOCAB = 100000
DIM = 128
B = 1024
L = 50

NC = 2
NS = 16
NW = NC * NS
RPW = B // NW
NLANE = DIM // 16


def _sc_pool_body(idx_hbm, table_hbm, out_hbm, idx_v, rows_v, out_v, sem):
    wid = lax.axis_index("s") * NC + lax.axis_index("c")
    pltpu.sync_copy(idx_hbm.at[wid], idx_v)

    def row_body(r, carry):
        pltpu.async_copy(table_hbm.at[idx_v.at[r]], rows_v, sem).wait()

        def lane_acc(l, accs):
            return tuple(accs[d] + rows_v[l, pl.ds(d * 16, 16)]
                         for d in range(NLANE))

        accs = tuple(rows_v[0, pl.ds(d * 16, 16)] for d in range(NLANE))
        accs = lax.fori_loop(1, L, lane_acc, accs)
        for d in range(NLANE):
            out_v[r, pl.ds(d * 16, 16)] = accs[d] * (1.0 / L)
        return carry

    lax.fori_loop(0, RPW, row_body, 0)
    pltpu.sync_copy(out_v, out_hbm.at[pl.ds(wid * RPW, RPW)])


_sc_pool = functools.partial(
    pl.kernel,
    out_type=jax.ShapeDtypeStruct((B, DIM), jnp.float32),
    mesh=plsc.VectorSubcoreMesh(core_axis_name="c", subcore_axis_name="s"),
    scratch_types=[
        pltpu.VMEM((RPW, L), jnp.int32),
        pltpu.VMEM((L, DIM), jnp.float32),
        pltpu.VMEM((RPW, DIM), jnp.float32),
        pltpu.SemaphoreType.DMA,
    ],
)(_sc_pool_body)


RB = 32
NRB = B // RB


def _mm_body(p_ref, w_ref, b_ref, o_ref):
    acc = lax.dot_general(p_ref[...], w_ref[...],
                          (((1,), (1,)), ((), ())),
                          preferred_element_type=jnp.float32)
    o_ref[...] = acc + b_ref[...]


def _decoder(pooled_bf, W_bf, b2d):
    return pl.pallas_call(
        _mm_body,
        grid=(NRB,),
        in_specs=[
            pl.BlockSpec((RB, DIM), lambda i: (i, 0)),
            pl.BlockSpec(memory_space=pltpu.VMEM),
            pl.BlockSpec(memory_space=pltpu.VMEM),
        ],
        out_specs=pl.BlockSpec((RB, VOCAB), lambda i: (i, 0)),
        out_shape=jax.ShapeDtypeStruct((B, VOCAB), jnp.float32),
        compiler_params=pltpu.CompilerParams(needs_layout_passes=True),
    )(pooled_bf, W_bf, b2d)


def kernel(contextTsr, emb_table, W, b):
    idx = contextTsr.reshape(NW, RPW, L)
    pooled = _sc_pool(idx, emb_table)
    return _decoder(pooled.astype(jnp.bfloat16), W.astype(jnp.bfloat16),
                    b.reshape(1, VOCAB))


# speedup vs baseline: 2.9720x; 2.9720x over previous
import functools

import jax
import jax.numpy as jnp
from jax import lax
from jax.experimental import pallas as pl
from jax.experimental.pallas import tpu as pltpu
from jax.experimental.pallas import tpu_sc as plsc

VOCAB = 100000
DIM = 128
B = 1024
L = 50

NC = 2
NS = 16
NW = NC * NS
RPW = B // NW
NLANE = DIM // 16


def _sc_pool_body(idx_hbm, table_hbm, out_hbm, idx_v, rows_v, out_v, sem):
    wid = lax.axis_index("s") * NC + lax.axis_index("c")
    pltpu.sync_copy(idx_hbm.at[wid], idx_v)

    def row_body(r, carry):
        pltpu.async_copy(table_hbm.at[idx_v.at[r]], rows_v, sem).wait()

        def lane_acc(l, accs):
            return tuple(accs[d] + rows_v[l, pl.ds(d * 16, 16)]
                         for d in range(NLANE))

        accs = tuple(rows_v[0, pl.ds(d * 16, 16)] for d in range(NLANE))
        accs = lax.fori_loop(1, L, lane_acc, accs)
        for d in range(NLANE):
            out_v[r, pl.ds(d * 16, 16)] = accs[d] * (1.0 / L)
        return carry

    lax.fori_loop(0, RPW, row_body, 0)
    pltpu.sync_copy(out_v, out_hbm.at[pl.ds(wid * RPW, RPW)])


_sc_pool = functools.partial(
    pl.kernel,
    out_type=jax.ShapeDtypeStruct((B, DIM), jnp.float32),
    mesh=plsc.VectorSubcoreMesh(core_axis_name="c", subcore_axis_name="s"),
    scratch_types=[
        pltpu.VMEM((RPW, L), jnp.int32),
        pltpu.VMEM((L, DIM), jnp.float32),
        pltpu.VMEM((RPW, DIM), jnp.float32),
        pltpu.SemaphoreType.DMA,
    ],
)(_sc_pool_body)


VBLK = 1024
NBLK = pl.cdiv(VOCAB, VBLK)


def _mm_body(w_ref, pt_ref, b_ref, o_ref):
    acc = lax.dot_general(w_ref[...], pt_ref[...],
                          (((1,), (0,)), ((), ())),
                          preferred_element_type=jnp.float32)
    o_ref[...] = acc + b_ref[...]


def _decoder(pooled_t, W, b2d):
    out_t = pl.pallas_call(
        _mm_body,
        grid=(NBLK,),
        in_specs=[
            pl.BlockSpec((VBLK, DIM), lambda i: (i, 0)),
            pl.BlockSpec((DIM, B), lambda i: (0, 0)),
            pl.BlockSpec((VBLK, 1), lambda i: (i, 0)),
        ],
        out_specs=pl.BlockSpec((VBLK, B), lambda i: (i, 0)),
        out_shape=jax.ShapeDtypeStruct((VOCAB, B), jnp.float32),
    )(W, pooled_t, b2d)
    return out_t.T


def kernel(contextTsr, emb_table, W, b):
    idx = contextTsr.reshape(NW, RPW, L)
    pooled = _sc_pool(idx, emb_table)
    return _decoder(pooled.T, W, b.reshape(VOCAB, 1))
